# Initial kernel scaffold; baseline (speedup 1.0000x reference)
import jax
import jax.numpy as jnp
from jax.experimental import pallas as pl


def kernel(inputs, W_in, b_in, W_out1, b_out1, W_out2, b_out2):
    raise NotImplementedError("write your pallas kernel here")


# speedup vs baseline: 16.4567x; 16.4567x over previous
import numpy as np
import jax
import jax.numpy as jnp
from jax.experimental import pallas as pl
from jax.experimental.pallas import tpu as pltpu

_DILATIONS = [1, 2, 1, 2]
_HIS_LEN = 12


def _composed_avg_matrix():
    A = np.eye(_HIS_LEN, dtype=np.float64)
    size = _HIS_LEN
    for d in _DILATIONS:
        m = size - d
        L = np.zeros((m, size))
        for i in range(m):
            L[i, i] = 0.5
            L[i, i + d] = 0.5
        A = L @ A
        size = m
    return A.astype(np.float32)


_A = _composed_avg_matrix()
_T = _HIS_LEN
_OUT_LEN = 12


def _fused_kernel(m_ref, bias_ref, x_ref, w_in_ref, w_out2_ref, b_out2_ref,
                  out_ref):
    x = x_ref[0]
    w_in = w_in_ref[...]
    w_out2 = w_out2_ref[...]
    b_out2 = b_out2_ref[...]
    for o in range(_OUT_LEN):
        z = m_ref[o, 0] * x[0]
        for t in range(1, _T):
            z = z + m_ref[o, t] * x[t]
        h = jnp.dot(z, w_in, preferred_element_type=jnp.float32)
        h = jnp.maximum(h + bias_ref[o], 0.0)
        y = jnp.dot(h, w_out2, preferred_element_type=jnp.float32)
        out_ref[0, o] = y + b_out2


def kernel(inputs, W_in, b_in, W_out1, b_out1, W_out2, b_out2):
    B, T, N, F = inputs.shape
    HID = W_in.shape[1]
    OUT_DIM = W_out2.shape[1]
    M = W_out1.T @ jnp.asarray(_A)
    bias = jnp.sum(M, axis=1, keepdims=True) * b_in[None, :] \
        + b_out1[:, None]

    out = pl.pallas_call(
        _fused_kernel,
        grid=(B,),
        in_specs=[
            pl.BlockSpec(memory_space=pltpu.SMEM),
            pl.BlockSpec((_OUT_LEN, HID), lambda b: (0, 0)),
            pl.BlockSpec((1, T, N, F), lambda b: (b, 0, 0, 0)),
            pl.BlockSpec((F, HID), lambda b: (0, 0)),
            pl.BlockSpec((HID, OUT_DIM), lambda b: (0, 0)),
            pl.BlockSpec((1, OUT_DIM), lambda b: (0, 0)),
        ],
        out_specs=pl.BlockSpec((1, _OUT_LEN, N, OUT_DIM),
                               lambda b: (b, 0, 0, 0)),
        out_shape=jax.ShapeDtypeStruct((B, _OUT_LEN, N, OUT_DIM),
                                       jnp.float32),
        compiler_params=pltpu.CompilerParams(
            dimension_semantics=("parallel",)),
    )(M, bias, inputs, W_in, W_out2, b_out2[None, :])
    return out
